# Initial kernel scaffold; baseline (speedup 1.0000x reference)
import functools

import numpy as np
import jax
import jax.numpy as jnp
from jax import lax
from jax.experimental import pallas as pl
from jax.experimental.pallas import tpu as pltpu
from jax.experimental.pallas import tpu_sc as plsc

_BN_EPS = 1e-5
_NW = 32


def _round_up(x, m):
    return (x + m - 1) // m * m


def _fps_body(x_ref, out_ref, *, npoint, N):
    xyz = x_ref[...]
    iota = lax.broadcasted_iota(jnp.float32, (1, N), 1)
    col_i = lax.broadcasted_iota(jnp.int32, (8, npoint), 1)

    def body(i, state):
        dist, farthest, cents = state
        sel = iota == farthest
        c = jnp.sum(jnp.where(sel, xyz, 0.0), axis=1, keepdims=True)
        cents = jnp.where(col_i == i, c, cents)
        dd = xyz - c
        dd = dd * dd
        dsum = (dd[0:1] + dd[1:2]) + dd[2:3]
        dist = jnp.minimum(dist, dsum)
        m = jnp.max(dist)
        far = jnp.min(jnp.where(dist == m, iota, jnp.float32(N)))
        return dist, far, cents

    dist0 = jnp.full((1, N), 1e10, jnp.float32)
    cents0 = jnp.zeros((8, npoint), jnp.float32)
    _, _, cents = lax.fori_loop(0, npoint, body, (dist0, jnp.float32(0.0), cents0))
    out_ref[...] = cents


def _fps(x8, npoint):
    B, _, N = x8.shape
    return pl.pallas_call(
        functools.partial(_fps_body, npoint=npoint, N=N),
        grid=(B,),
        in_specs=[pl.BlockSpec((None, 8, N), lambda b: (b, 0, 0))],
        out_specs=pl.BlockSpec((None, 8, npoint), lambda b: (b, 0, 0)),
        out_shape=jax.ShapeDtypeStruct((B, 8, npoint), jnp.float32),
    )(x8)


def _ballquery_body(nx_ref, x_ref, o1_ref, o2_ref, *, N, Ks, rsqs):
    c = nx_ref[...]
    p = x_ref[...]
    cs2 = jnp.sum(c * c, axis=1, keepdims=True)
    ps2 = jnp.sum(p * p, axis=0, keepdims=True)
    mm = lax.dot_general(c, p, (((1,), (0,)), ((), ())),
                         preferred_element_type=jnp.float32)
    d = (cs2 + ps2) - 2.0 * mm
    iota = lax.broadcasted_iota(jnp.float32, d.shape, 1)
    BIG = jnp.float32(3 * N)
    off = pl.program_id(0) * N
    for o_ref, K, rsq in ((o1_ref, Ks[0], rsqs[0]), (o2_ref, Ks[1], rsqs[1])):
        col_k = lax.broadcasted_iota(jnp.int32, (d.shape[0], K), 1)
        v = jnp.where(d > rsq, BIG, iota)
        m0 = jnp.min(v, axis=1, keepdims=True)
        acc = jnp.broadcast_to(m0, (d.shape[0], K))
        prev = m0
        for k in range(1, K):
            v = jnp.where(v == prev, BIG, v)
            m = jnp.min(v, axis=1, keepdims=True)
            mval = jnp.where(m < N, m, m0)
            acc = jnp.where(col_k == k, mval, acc)
            prev = m
        o_ref[...] = acc.astype(jnp.int32) + off


def _ballquery(nxr, x8, radii, nsamples, s_blk):
    B, S, _ = nxr.shape
    N = x8.shape[2]
    K1, K2 = nsamples
    rsqs = (np.float32(radii[0] ** 2), np.float32(radii[1] ** 2))
    return pl.pallas_call(
        functools.partial(_ballquery_body, N=N, Ks=(K1, K2), rsqs=rsqs),
        grid=(B, S // s_blk),
        in_specs=[pl.BlockSpec((None, s_blk, 8), lambda b, s: (b, s, 0)),
                  pl.BlockSpec((None, 8, N), lambda b, s: (b, 0, 0))],
        out_specs=[pl.BlockSpec((None, s_blk, K1), lambda b, s: (b, s, 0)),
                   pl.BlockSpec((None, s_blk, K2), lambda b, s: (b, s, 0))],
        out_shape=[jax.ShapeDtypeStruct((B, S, K1), jnp.int32),
                   jax.ShapeDtypeStruct((B, S, K2), jnp.int32)],
    )(nxr, x8)


def _sc_gather(table2d, idx2d, R, D, G):
    b_per_w = R // _NW
    n_chunks = b_per_w // G
    mesh = plsc.VectorSubcoreMesh(core_axis_name="c", subcore_axis_name="s")

    @functools.partial(
        pl.kernel,
        out_type=jax.ShapeDtypeStruct((R, D), jnp.float32),
        mesh=mesh,
        scratch_types=[
            pltpu.VMEM((n_chunks, G), jnp.int32),
            pltpu.VMEM((b_per_w, D), jnp.float32),
            pltpu.SemaphoreType.DMA,
        ],
    )
    def gather_kernel(table_hbm, idx_hbm, out_hbm, idx_v, rows_v, sem):
        wid = lax.axis_index("s") * 2 + lax.axis_index("c")
        row0 = wid * n_chunks
        pltpu.sync_copy(idx_hbm.at[pl.ds(row0, n_chunks)], idx_v)
        copies = []
        for j in range(n_chunks):
            copies.append(
                pltpu.async_copy(table_hbm.at[idx_v.at[j]],
                                 rows_v.at[pl.ds(j * G, G)], sem))
        for cp in copies:
            cp.wait()
        pltpu.sync_copy(rows_v, out_hbm.at[pl.ds(wid * b_per_w, b_per_w)])

    return gather_kernel(table2d, idx2d)


def _sa_mlp_body(*refs, K, n_layers):
    g_ref, c_ref, p_ref = refs[0], refs[1], refs[2]
    out_ref = refs[3 + 2 * n_layers]
    g = g_ref[...]
    c = c_ref[...]
    P = p_ref[...]
    cpad = lax.dot_general(c, P, (((1,), (0,)), ((), ())),
                           preferred_element_type=jnp.float32)
    h = g - cpad
    for li in range(n_layers):
        W = refs[3 + 2 * li][...]
        b = refs[4 + 2 * li][...]
        h = lax.dot_general(h, W, (((1,), (0,)), ((), ())),
                            preferred_element_type=jnp.float32)
        h = jnp.maximum(h + b, 0.0)
    w_out = h.shape[1]
    h = h.reshape(h.shape[0] // K, K, w_out)
    out_ref[...] = jnp.max(h, axis=1)


def _sa_mlp(g3, cexp, layers, C, D, K, s_blk):
    B = g3.shape[0]
    S = g3.shape[1] // K
    P = np.zeros((8, D), np.float32)
    for i in range(3):
        P[i, C + i] = 1.0
    w_out = layers[-1][0].shape[1]
    args = [g3, cexp, jnp.asarray(P)]
    in_specs = [pl.BlockSpec((None, s_blk * K, D), lambda b, s: (b, s, 0)),
                pl.BlockSpec((None, s_blk * K, 8), lambda b, s: (b, s, 0)),
                pl.BlockSpec((8, D), lambda b, s: (0, 0))]
    for Wp, bp in layers:
        args += [Wp, bp]
        in_specs += [pl.BlockSpec(Wp.shape, lambda b, s: (0, 0)),
                     pl.BlockSpec(bp.shape, lambda b, s: (0, 0))]
    return pl.pallas_call(
        functools.partial(_sa_mlp_body, K=K, n_layers=len(layers)),
        grid=(B, S // s_blk),
        in_specs=in_specs,
        out_specs=pl.BlockSpec((None, s_blk, w_out), lambda b, s: (b, s, 0)),
        out_shape=jax.ShapeDtypeStruct((B, S, w_out), jnp.float32),
    )(*args)


def _fp_body(*refs, S, n_layers, has_p1):
    x1 = refs[0][...]
    x2 = refs[1][...]
    p2 = refs[2][...]
    i = 3
    if has_p1:
        p1 = refs[i][...]
        i += 1
    wrefs = refs[i:i + 2 * n_layers + (1 if has_p1 else 0)]
    out_ref = refs[-1]

    cs2 = jnp.sum(x1 * x1, axis=1, keepdims=True)
    ps2 = jnp.sum(x2 * x2, axis=0, keepdims=True)
    mm = lax.dot_general(x1, x2, (((1,), (0,)), ((), ())),
                         preferred_element_type=jnp.float32)
    d = (cs2 + ps2) - 2.0 * mm
    iota = lax.broadcasted_iota(jnp.float32, d.shape, 1)
    v = d
    ms, sels = [], []
    for t in range(3):
        m = jnp.min(v, axis=1, keepdims=True)
        j = jnp.min(jnp.where(v == m, iota, jnp.float32(S)), axis=1,
                    keepdims=True)
        sel = iota == j
        ms.append(m)
        sels.append(sel)
        if t < 2:
            v = jnp.where(sel, jnp.float32(1e30), v)
    r = [1.0 / (m + 1e-8) for m in ms]
    norm = (r[0] + r[1]) + r[2]
    A = (jnp.where(sels[0], r[0] / norm, 0.0)
         + jnp.where(sels[1], r[1] / norm, 0.0)
         + jnp.where(sels[2], r[2] / norm, 0.0))
    interp = lax.dot_general(A, p2, (((1,), (0,)), ((), ())),
                             preferred_element_type=jnp.float32)

    wi = 0
    if has_p1:
        Wa = wrefs[wi][...]
        wi += 1
        h = lax.dot_general(p1, Wa, (((1,), (0,)), ((), ())),
                            preferred_element_type=jnp.float32)
    else:
        h = 0.0
    Wb = wrefs[wi][...]
    b1 = wrefs[wi + 1][...]
    wi += 2
    h = h + lax.dot_general(interp, Wb, (((1,), (0,)), ((), ())),
                            preferred_element_type=jnp.float32)
    h = jnp.maximum(h + b1, 0.0)
    for li in range(1, n_layers):
        W = wrefs[wi][...]
        b = wrefs[wi + 1][...]
        wi += 2
        h = lax.dot_general(h, W, (((1,), (0,)), ((), ())),
                            preferred_element_type=jnp.float32)
        h = jnp.maximum(h + b, 0.0)
    out_ref[...] = h


def _fp(xr1, x82, pts1, pts2, layers, n_blk):
    B, N, _ = xr1.shape
    S = x82.shape[2]
    C2 = pts2.shape[2]
    has_p1 = pts1 is not None
    args = [xr1, x82, pts2]
    in_specs = [pl.BlockSpec((None, n_blk, 8), lambda b, n: (b, n, 0)),
                pl.BlockSpec((None, 8, S), lambda b, n: (b, 0, 0)),
                pl.BlockSpec((None, S, C2), lambda b, n: (b, 0, 0))]
    if has_p1:
        C1 = pts1.shape[2]
        args.append(pts1)
        in_specs.append(pl.BlockSpec((None, n_blk, C1), lambda b, n: (b, n, 0)))
    W1, b1 = layers[0]
    if has_p1:
        warr = [W1[:C1], W1[C1:], b1]
    else:
        warr = [W1, b1]
    for W, b in layers[1:]:
        warr += [W, b]
    for w in warr:
        args.append(w)
        in_specs.append(pl.BlockSpec(w.shape, lambda b, n: (0, 0)))
    w_out = layers[-1][0].shape[1]
    return pl.pallas_call(
        functools.partial(_fp_body, S=S, n_layers=len(layers), has_p1=has_p1),
        grid=(B, N // n_blk),
        in_specs=in_specs,
        out_specs=pl.BlockSpec((None, n_blk, w_out), lambda b, n: (b, n, 0)),
        out_shape=jax.ShapeDtypeStruct((B, N, w_out), jnp.float32),
    )(*args)


def _fold(layer):
    W, b, gamma, beta = layer
    s = gamma / jnp.sqrt(1.0 + _BN_EPS)
    Wp = jnp.transpose(W * s[:, None])
    bp = (b * s + beta)[None, :]
    return Wp, bp


def _sa_level(x8, xr, pts, cfg, scale_params):
    npoint, radii, nsamples, in_ch = cfg
    B, _, N = x8.shape
    S = npoint
    nx8 = _fps(x8, S)
    nxr = jnp.transpose(nx8, (0, 2, 1))
    s_blk = min(S, 256)
    gidx1, gidx2 = _ballquery(nxr, x8, radii, nsamples, s_blk)
    C = in_ch
    D = _round_up(C + 3, 16)
    if pts is None:
        table = jnp.concatenate(
            [xr[..., :3], jnp.zeros((B, N, D - 3), jnp.float32)], axis=-1)
    else:
        table = jnp.concatenate(
            [pts, xr[..., :3], jnp.zeros((B, N, D - C - 3), jnp.float32)],
            axis=-1)
    table2d = table.reshape(B * N, D)
    outs = []
    for i, (gidx, K) in enumerate(zip((gidx1, gidx2), nsamples)):
        R = B * S * K
        G = min(R // _NW, 128)
        idx2d = gidx.reshape(R // G, G)
        rows = _sc_gather(table2d, idx2d, R, D, G)
        g3 = rows.reshape(B, S * K, D)
        cexp = jnp.broadcast_to(nxr[:, :, None, :],
                                (B, S, K, 8)).reshape(B, S * K, 8)
        layers = [_fold(l) for l in scale_params[i]]
        layers[0] = (jnp.concatenate(
            [layers[0][0],
             jnp.zeros((D - (C + 3), layers[0][0].shape[1]), jnp.float32)],
            axis=0), layers[0][1])
        outs.append(_sa_mlp(g3, cexp, layers, C, D, K, s_blk))
    return nx8, nxr, jnp.concatenate(outs, axis=-1)


_SA_CFG = [
    (1024, (0.05, 0.1), (16, 32), 0),
    (256, (0.1, 0.2), (16, 32), 96),
    (64, (0.2, 0.4), (16, 32), 256),
    (16, (0.4, 0.8), (16, 32), 512),
]
_FP_NBLK = [64, 256, 256, 512]


def kernel(xyz, params):
    B, _, N0 = xyz.shape
    x8 = jnp.concatenate([xyz, jnp.zeros((B, 5, N0), jnp.float32)], axis=1)
    xr = jnp.transpose(x8, (0, 2, 1))

    x8s, xrs, ptss = [x8], [xr], [None]
    for lvl in range(4):
        nx8, nxr, npts = _sa_level(x8s[-1], xrs[-1], ptss[-1], _SA_CFG[lvl],
                                   params["sa"][lvl])
        x8s.append(nx8)
        xrs.append(nxr)
        ptss.append(npts)

    pts = list(ptss)
    for fi, (dst, src) in enumerate(((3, 4), (2, 3), (1, 2), (0, 1))):
        layers = [_fold(l) for l in params["fp"][fi]]
        pts[dst] = _fp(xrs[dst], x8s[src], pts[dst], pts[src], layers,
                       _FP_NBLK[fi])

    return jnp.transpose(pts[0], (0, 2, 1))


# speedup vs baseline: 11.0990x; 11.0990x over previous
import functools

import numpy as np
import jax
import jax.numpy as jnp
from jax import lax
from jax.experimental import pallas as pl
from jax.experimental.pallas import tpu as pltpu
from jax.experimental.pallas import tpu_sc as plsc

_BN_EPS = 1e-5
_BN_DENOM = np.float32(np.sqrt(np.float32(1.0 + _BN_EPS)))
_NW = 32


def _bdot(a, b):
    return lax.dot_general(a.astype(jnp.bfloat16), b.astype(jnp.bfloat16),
                           (((1,), (0,)), ((), ())),
                           preferred_element_type=jnp.float32)


def _round_up(x, m):
    return (x + m - 1) // m * m


def _fps_body(x_ref, out_ref, *, npoint, N):
    xyz = x_ref[...]
    iota = lax.broadcasted_iota(jnp.int32, (1, N), 1).astype(jnp.float32)
    col_i = lax.broadcasted_iota(jnp.int32, (8, npoint), 1)

    def body(i, state):
        dist, farthest, cents = state
        sel = iota == farthest
        c = jnp.sum(jnp.where(sel, xyz, 0.0), axis=1, keepdims=True)
        cents = jnp.where(col_i == i, c, cents)
        dd = xyz - c
        dd = dd * dd
        dsum = (dd[0:1] + dd[1:2]) + dd[2:3]
        dist = jnp.minimum(dist, dsum)
        m = jnp.max(dist)
        far = jnp.min(jnp.where(dist == m, iota, jnp.float32(N)))
        return dist, far, cents

    dist0 = jnp.full((1, N), 1e10, jnp.float32)
    cents0 = jnp.zeros((8, npoint), jnp.float32)
    _, _, cents = lax.fori_loop(0, npoint, body, (dist0, jnp.float32(0.0), cents0))
    out_ref[...] = cents


def _fps(x8, npoint):
    B, _, N = x8.shape
    return pl.pallas_call(
        functools.partial(_fps_body, npoint=npoint, N=N),
        grid=(B,),
        in_specs=[pl.BlockSpec((None, 8, N), lambda b: (b, 0, 0))],
        out_specs=pl.BlockSpec((None, 8, npoint), lambda b: (b, 0, 0)),
        out_shape=jax.ShapeDtypeStruct((B, 8, npoint), jnp.float32),
    )(x8)


def _ballquery_body(nx_ref, x_ref, o1_ref, o2_ref, *, N, Ks, rsqs):
    c = nx_ref[...]
    p = x_ref[...]
    cs2 = jnp.sum(c * c, axis=1, keepdims=True)
    ps2 = jnp.sum(p * p, axis=0, keepdims=True)
    mm = _bdot(c, p)
    d = (cs2 + ps2) - 2.0 * mm
    iota = lax.broadcasted_iota(jnp.int32, d.shape, 1).astype(jnp.float32)
    BIG = jnp.float32(3 * N)
    off = pl.program_id(0) * N
    for o_ref, K, rsq in ((o1_ref, Ks[0], rsqs[0]), (o2_ref, Ks[1], rsqs[1])):
        col_k = lax.broadcasted_iota(jnp.int32, (d.shape[0], K), 1)
        v = jnp.where(d > rsq, BIG, iota)
        m0 = jnp.min(v, axis=1, keepdims=True)
        acc = jnp.broadcast_to(m0, (d.shape[0], K))
        prev = m0
        for k in range(1, K):
            v = jnp.where(v == prev, BIG, v)
            m = jnp.min(v, axis=1, keepdims=True)
            mval = jnp.where(m < N, m, m0)
            acc = jnp.where(col_k == k, mval, acc)
            prev = m
        acc = jnp.minimum(acc, jnp.float32(N - 1))
        o_ref[...] = acc.astype(jnp.int32) + off


def _ballquery(nxr, x8, radii, nsamples, s_blk):
    B, S, _ = nxr.shape
    N = x8.shape[2]
    K1, K2 = nsamples
    rsqs = (np.float32(radii[0] ** 2), np.float32(radii[1] ** 2))
    return pl.pallas_call(
        functools.partial(_ballquery_body, N=N, Ks=(K1, K2), rsqs=rsqs),
        grid=(B, S // s_blk),
        in_specs=[pl.BlockSpec((None, s_blk, 8), lambda b, s: (b, s, 0)),
                  pl.BlockSpec((None, 8, N), lambda b, s: (b, 0, 0))],
        out_specs=[pl.BlockSpec((None, s_blk, K1), lambda b, s: (b, s, 0)),
                   pl.BlockSpec((None, s_blk, K2), lambda b, s: (b, s, 0))],
        out_shape=[jax.ShapeDtypeStruct((B, S, K1), jnp.int32),
                   jax.ShapeDtypeStruct((B, S, K2), jnp.int32)],
    )(nxr, x8)


def _sc_gather(table2d, idx2d, R, D, G):
    b_per_w = R // _NW
    n_chunks = b_per_w // G
    mesh = plsc.VectorSubcoreMesh(core_axis_name="c", subcore_axis_name="s")

    if b_per_w * D * 4 <= 300_000:
        @functools.partial(
            pl.kernel,
            out_type=jax.ShapeDtypeStruct((R, D), jnp.float32),
            mesh=mesh,
            scratch_types=[
                pltpu.VMEM((n_chunks, G), jnp.int32),
                pltpu.VMEM((b_per_w, D), jnp.float32),
                pltpu.SemaphoreType.DMA,
            ],
            compiler_params=pltpu.CompilerParams(use_tc_tiling_on_sc=False),
        )
        def gather_kernel(table_hbm, idx_hbm, out_hbm, idx_v, rows_v, sem):
            wid = lax.axis_index("s") * 2 + lax.axis_index("c")
            row0 = wid * n_chunks
            pltpu.sync_copy(idx_hbm.at[pl.ds(row0, n_chunks)], idx_v)
            copies = []
            for j in range(n_chunks):
                copies.append(
                    pltpu.async_copy(table_hbm.at[idx_v.at[j]],
                                     rows_v.at[pl.ds(j * G, G)], sem))
            for cp in copies:
                cp.wait()
            pltpu.sync_copy(rows_v, out_hbm.at[pl.ds(wid * b_per_w, b_per_w)])

        return gather_kernel(table2d, idx2d)

    @functools.partial(
        pl.kernel,
        out_type=jax.ShapeDtypeStruct((R, D), jnp.float32),
        mesh=mesh,
        scratch_types=[
            pltpu.VMEM((n_chunks, G), jnp.int32),
            pltpu.VMEM((G, D), jnp.float32),
            pltpu.VMEM((G, D), jnp.float32),
            pltpu.SemaphoreType.DMA,
            pltpu.SemaphoreType.DMA,
        ],
        compiler_params=pltpu.CompilerParams(use_tc_tiling_on_sc=False),
    )
    def gather_stream(table_hbm, idx_hbm, out_hbm, idx_v, buf0, buf1,
                      sem0, sem1):
        wid = lax.axis_index("s") * 2 + lax.axis_index("c")
        row0 = wid * n_chunks
        base = wid * b_per_w
        pltpu.sync_copy(idx_hbm.at[pl.ds(row0, n_chunks)], idx_v)
        bufs = (buf0, buf1)
        sems = (sem0, sem1)
        prev = None
        for j in range(n_chunks):
            cur = pltpu.async_copy(table_hbm.at[idx_v.at[j]], bufs[j % 2],
                                   sems[j % 2])
            if prev is not None:
                pj, pcp = prev
                pcp.wait()
                pltpu.sync_copy(bufs[pj % 2],
                                out_hbm.at[pl.ds(base + pj * G, G)])
            prev = (j, cur)
        pj, pcp = prev
        pcp.wait()
        pltpu.sync_copy(bufs[pj % 2], out_hbm.at[pl.ds(base + pj * G, G)])

    return gather_stream(table2d, idx2d)


def _sa_mlp_body(*refs, K, n_layers, C):
    g_ref, c_ref = refs[0], refs[1]
    out_ref = refs[2 + 4 * n_layers]
    g = g_ref[...]
    c = c_ref[...]
    if C > 0:
        h = jnp.concatenate(
            [g[:, :C], g[:, C:C + 3] - c[:, :3], g[:, C + 3:]], axis=1)
    else:
        h = jnp.concatenate([g[:, :3] - c[:, :3], g[:, 3:]], axis=1)
    for li in range(n_layers):
        W = refs[2 + 4 * li][...]
        b = refs[3 + 4 * li][...]
        gam = refs[4 + 4 * li][...]
        bet = refs[5 + 4 * li][...]
        h = _bdot(h, W) + b
        h = h / _BN_DENOM
        h = jnp.maximum(h * gam + bet, 0.0)
    w_out = h.shape[1]
    h = h.reshape(h.shape[0] // K, K, w_out)
    out_ref[...] = jnp.max(h, axis=1)


def _sa_mlp(g3, cexp, layers, C, D, K, s_blk):
    B = g3.shape[0]
    S = g3.shape[1] // K
    w_out = layers[-1][0].shape[1]
    args = [g3, cexp]
    in_specs = [pl.BlockSpec((None, s_blk * K, D), lambda b, s: (b, s, 0)),
                pl.BlockSpec((None, s_blk * K, 8), lambda b, s: (b, s, 0))]
    for larr in layers:
        for w in larr:
            args.append(w)
            in_specs.append(pl.BlockSpec(w.shape, lambda b, s: (0, 0)))
    return pl.pallas_call(
        functools.partial(_sa_mlp_body, K=K, n_layers=len(layers), C=C),
        grid=(B, S // s_blk),
        in_specs=in_specs,
        out_specs=pl.BlockSpec((None, s_blk, w_out), lambda b, s: (b, s, 0)),
        out_shape=jax.ShapeDtypeStruct((B, S, w_out), jnp.float32),
    )(*args)


def _fp_knn_body(x1_ref, x2_ref, oi_ref, ow_ref, *, S):
    x1 = x1_ref[...]
    x2 = x2_ref[...]
    cs2 = jnp.sum(x1 * x1, axis=1, keepdims=True)
    ps2 = jnp.sum(x2 * x2, axis=0, keepdims=True)
    mm = _bdot(x1, x2)
    d = (cs2 + ps2) - 2.0 * mm
    iota = lax.broadcasted_iota(jnp.int32, d.shape, 1).astype(jnp.float32)
    v = d
    ms, js = [], []
    for t in range(3):
        m = jnp.min(v, axis=1, keepdims=True)
        j = jnp.min(jnp.where(v == m, iota, jnp.float32(S)), axis=1,
                    keepdims=True)
        ms.append(m)
        js.append(j)
        if t < 2:
            v = jnp.where(iota == j, jnp.float32(1e30), v)
    r = [1.0 / (m + 1e-8) for m in ms]
    norm = (r[0] + r[1]) + r[2]
    w = [r[t] / norm for t in range(3)]
    col = lax.broadcasted_iota(jnp.int32, (d.shape[0], 4), 1)
    acc_i = jnp.broadcast_to(js[0], (d.shape[0], 4))
    acc_w = jnp.broadcast_to(w[0], (d.shape[0], 4))
    for t in (1, 2):
        acc_i = jnp.where(col == t, js[t], acc_i)
        acc_w = jnp.where(col == t, w[t], acc_w)
    acc_w = jnp.where(col == 3, 0.0, acc_w)
    off = pl.program_id(0) * S
    oi_ref[...] = acc_i.astype(jnp.int32) + off
    ow_ref[...] = acc_w


def _fp_knn(xr1, x82, n_blk):
    B, N, _ = xr1.shape
    S = x82.shape[2]
    return pl.pallas_call(
        functools.partial(_fp_knn_body, S=S),
        grid=(B, N // n_blk),
        in_specs=[pl.BlockSpec((None, n_blk, 8), lambda b, n: (b, n, 0)),
                  pl.BlockSpec((None, 8, S), lambda b, n: (b, 0, 0))],
        out_specs=[pl.BlockSpec((None, n_blk, 4), lambda b, n: (b, n, 0)),
                   pl.BlockSpec((None, n_blk, 4), lambda b, n: (b, n, 0))],
        out_shape=[jax.ShapeDtypeStruct((B, N, 4), jnp.int32),
                   jax.ShapeDtypeStruct((B, N, 4), jnp.float32)],
    )(xr1, x82)


def _fp_mlp_body(*refs, n_layers, has_p1):
    g_ref, w_ref = refs[0], refs[1]
    i = 2
    p1 = None
    if has_p1:
        p1 = refs[i][...]
        i += 1
    wrefs = refs[i:i + 4 * n_layers]
    out_ref = refs[-1]
    g = g_ref[...]
    wv = w_ref[...]
    n_blk = wv.shape[0]
    c2 = g.shape[1]
    g3 = g.reshape(n_blk, 4, c2)
    interp = ((g3[:, 0] * wv[:, 0:1] + g3[:, 1] * wv[:, 1:2])
              + g3[:, 2] * wv[:, 2:3])
    h = None
    for li in range(n_layers):
        W = wrefs[4 * li][...]
        b = wrefs[4 * li + 1][...]
        gam = wrefs[4 * li + 2][...]
        bet = wrefs[4 * li + 3][...]
        if li == 0 and has_p1:
            c1 = p1.shape[1]
            if c1 % 256 == 0:
                h = _bdot(jnp.concatenate([p1, interp], axis=1), W) + b
            else:
                h = (_bdot(p1, W[:c1]) + _bdot(interp, W[c1:])) + b
        elif li == 0:
            h = _bdot(interp, W) + b
        else:
            h = _bdot(h, W) + b
        h = h / _BN_DENOM
        h = jnp.maximum(h * gam + bet, 0.0)
    out_ref[...] = h


def _fp(xr1, x82, pts1, pts2, layers, n_blk):
    B, N, _ = xr1.shape
    S = x82.shape[2]
    C2 = pts2.shape[2]
    has_p1 = pts1 is not None
    idx, wts = _fp_knn(xr1, x82, n_blk)
    R = B * N * 4
    G = min(R // _NW, 128)
    while (R // _NW) % G:
        G //= 2
    rows = _sc_gather(pts2.reshape(B * S, C2), idx.reshape(R // G, G),
                      R, C2, G)
    g = rows.reshape(B, N * 4, C2)

    args = [g, wts]
    in_specs = [pl.BlockSpec((None, n_blk * 4, C2), lambda b, n: (b, n, 0)),
                pl.BlockSpec((None, n_blk, 4), lambda b, n: (b, n, 0))]
    if has_p1:
        C1 = pts1.shape[2]
        args.append(pts1)
        in_specs.append(pl.BlockSpec((None, n_blk, C1), lambda b, n: (b, n, 0)))
    for larr in layers:
        for w in larr:
            args.append(w)
            in_specs.append(pl.BlockSpec(w.shape, lambda b, n: (0, 0)))
    w_out = layers[-1][0].shape[1]
    return pl.pallas_call(
        functools.partial(_fp_mlp_body, n_layers=len(layers), has_p1=has_p1),
        grid=(B, N // n_blk),
        in_specs=in_specs,
        out_specs=pl.BlockSpec((None, n_blk, w_out), lambda b, n: (b, n, 0)),
        out_shape=jax.ShapeDtypeStruct((B, N, w_out), jnp.float32),
    )(*args)


def _prep(layer):
    W, b, gamma, beta = layer
    Wt = jnp.transpose(W).astype(jnp.bfloat16)
    return Wt, b[None, :], gamma[None, :], beta[None, :]


def _sa_level(x8, xr, pts, cfg, scale_params):
    npoint, radii, nsamples, in_ch = cfg
    B, _, N = x8.shape
    S = npoint
    nx8 = _fps(x8, S)
    nxr = jnp.transpose(nx8, (0, 2, 1))
    s_blk = min(S, 256)
    gidx1, gidx2 = _ballquery(nxr, x8, radii, nsamples, s_blk)
    C = in_ch
    D = _round_up(C + 3, 16)
    if pts is None:
        table = jnp.concatenate(
            [xr[..., :3], jnp.zeros((B, N, D - 3), jnp.float32)], axis=-1)
    else:
        table = jnp.concatenate(
            [pts, xr[..., :3], jnp.zeros((B, N, D - C - 3), jnp.float32)],
            axis=-1)
    table2d = table.reshape(B * N, D)
    outs = []
    for i, (gidx, K) in enumerate(zip((gidx1, gidx2), nsamples)):
        R = B * S * K
        G = min(R // _NW, 128)
        idx2d = gidx.reshape(R // G, G)
        rows = _sc_gather(table2d, idx2d, R, D, G)
        g3 = rows.reshape(B, S * K, D)
        cexp = jnp.broadcast_to(nxr[:, :, None, :],
                                (B, S, K, 8)).reshape(B, S * K, 8)
        layers = [_prep(l) for l in scale_params[i]]
        layers[0] = (jnp.concatenate(
            [layers[0][0],
             jnp.zeros((D - (C + 3), layers[0][0].shape[1]), jnp.bfloat16)],
            axis=0),) + layers[0][1:]
        outs.append(_sa_mlp(g3, cexp, layers, C, D, K, s_blk))
    return nx8, nxr, jnp.concatenate(outs, axis=-1)


_SA_CFG = [
    (1024, (0.05, 0.1), (16, 32), 0),
    (256, (0.1, 0.2), (16, 32), 96),
    (64, (0.2, 0.4), (16, 32), 256),
    (16, (0.4, 0.8), (16, 32), 512),
]
_FP_NBLK = [64, 256, 256, 512]


def kernel(xyz, params):
    B, _, N0 = xyz.shape
    x8 = jnp.concatenate([xyz, jnp.zeros((B, 5, N0), jnp.float32)], axis=1)
    xr = jnp.transpose(x8, (0, 2, 1))

    x8s, xrs, ptss = [x8], [xr], [None]
    for lvl in range(4):
        nx8, nxr, npts = _sa_level(x8s[-1], xrs[-1], ptss[-1], _SA_CFG[lvl],
                                   params["sa"][lvl])
        x8s.append(nx8)
        xrs.append(nxr)
        ptss.append(npts)

    pts = list(ptss)
    for fi, (dst, src) in enumerate(((3, 4), (2, 3), (1, 2), (0, 1))):
        layers = [_prep(l) for l in params["fp"][fi]]
        pts[dst] = _fp(xrs[dst], x8s[src], pts[dst], pts[src], layers,
                       _FP_NBLK[fi])

    return jnp.transpose(pts[0], (0, 2, 1))
